# Initial kernel scaffold; baseline (speedup 1.0000x reference)
import jax
import jax.numpy as jnp
from jax.experimental import pallas as pl


def kernel(conf_preds, loc_delta, anchors):
    raise NotImplementedError("write your pallas kernel here")


# speedup vs baseline: 2.1646x; 2.1646x over previous
import jax
import jax.numpy as jnp
from jax.experimental import pallas as pl
from jax.experimental.pallas import tpu as pltpu

_NUM_CLASSES = 21
_TOP_K = 200
_PRE_NMS = 500
_CONF_THRESH = 0.05
_IOU_THRESH = 0.5
_VAR0, _VAR1 = 0.1, 0.2
_CLIP_W = _CLIP_H = 1.0
_PRE_PAD = 512
_K_PAD = 256
_NEG = -1e30


def _nms_kernel(cs_ref, cb_ref, outs_ref, outb_ref, ws_ref):
    n, m = cs_ref.shape
    outs_ref[...] = jnp.zeros_like(outs_ref)
    outb_ref[...] = jnp.zeros_like(outb_ref)
    ws_ref[...] = cs_ref[...]
    x1 = cb_ref[0]
    y1 = cb_ref[1]
    x2 = cb_ref[2]
    y2 = cb_ref[3]
    area = jnp.maximum(x2 - x1, 0.0) * jnp.maximum(y2 - y1, 0.0)
    iota = jax.lax.broadcasted_iota(jnp.int32, (n, m), 1)

    def body(r, carry):
        ws = ws_ref[...]
        best = jnp.max(ws, axis=1, keepdims=True)
        eq = ws == best
        pos = jnp.min(jnp.where(eq, iota, m), axis=1, keepdims=True)
        chosen = iota == pos
        chf = chosen.astype(jnp.float32)
        bx1 = jnp.sum(x1 * chf, axis=1, keepdims=True)
        by1 = jnp.sum(y1 * chf, axis=1, keepdims=True)
        bx2 = jnp.sum(x2 * chf, axis=1, keepdims=True)
        by2 = jnp.sum(y2 * chf, axis=1, keepdims=True)
        barea = jnp.sum(area * chf, axis=1, keepdims=True)
        valid = best > _CONF_THRESH
        tile = pl.multiple_of((r // 128) * 128, 128)
        off = r - tile
        wmask = jax.lax.broadcasted_iota(jnp.int32, (1, 128), 1) == off
        for ref_slice, val in (
            (outs_ref.at[:, pl.ds(tile, 128)], jnp.where(valid, best, 0.0)),
            (outb_ref.at[0, :, pl.ds(tile, 128)], jnp.where(valid, bx1, 0.0)),
            (outb_ref.at[1, :, pl.ds(tile, 128)], jnp.where(valid, by1, 0.0)),
            (outb_ref.at[2, :, pl.ds(tile, 128)], jnp.where(valid, bx2, 0.0)),
            (outb_ref.at[3, :, pl.ds(tile, 128)], jnp.where(valid, by2, 0.0)),
        ):
            ref_slice[...] = jnp.where(wmask, val, ref_slice[...])
        ltx = jnp.maximum(x1, bx1)
        lty = jnp.maximum(y1, by1)
        rbx = jnp.minimum(x2, bx2)
        rby = jnp.minimum(y2, by2)
        iw = jnp.maximum(rbx - ltx, 0.0)
        ih = jnp.maximum(rby - lty, 0.0)
        inter = iw * ih
        union = area + barea - inter
        iou = inter / jnp.maximum(union, 1e-9)
        kill = chosen | ((iou > _IOU_THRESH) & valid)
        ws_ref[...] = jnp.where(kill, _NEG, ws)
        return carry

    jax.lax.fori_loop(0, _TOP_K, body, 0)


def _run_nms(cs, cb):
    n = cs.shape[0]
    outs, outb = pl.pallas_call(
        _nms_kernel,
        out_shape=[
            jax.ShapeDtypeStruct((n, _K_PAD), jnp.float32),
            jax.ShapeDtypeStruct((4, n, _K_PAD), jnp.float32),
        ],
        scratch_shapes=[pltpu.VMEM((n, _PRE_PAD), jnp.float32)],
    )(cs, cb)
    return outs, outb


def _decode_clip(loc_delta, anchors):
    anch = anchors[None, :, :]
    cxcy = anch[..., :2] + loc_delta[..., :2] * _VAR0 * anch[..., 2:]
    wh = anch[..., 2:] * jnp.exp(loc_delta[..., 2:] * _VAR1)
    boxes = jnp.concatenate([cxcy - wh / 2.0, cxcy + wh / 2.0], axis=-1)
    x = jnp.clip(boxes[..., 0::2], 0.0, _CLIP_W)
    y = jnp.clip(boxes[..., 1::2], 0.0, _CLIP_H)
    return jnp.stack([x[..., 0], y[..., 0], x[..., 1], y[..., 1]], axis=-1)


def kernel(conf_preds, loc_delta, anchors):
    nb, na, nc = conf_preds.shape
    decoded = _decode_clip(loc_delta, anchors)
    conf_t = conf_preds.transpose(0, 2, 1)
    w = decoded[..., 2] - decoded[..., 0]
    h = decoded[..., 3] - decoded[..., 1]
    validb = (w >= 0.0) & (h >= 0.0)
    scores = jnp.where(validb[:, None, :], conf_t, -jnp.inf)

    vals, idx = jax.lax.top_k(scores, _PRE_NMS)
    cb = jnp.take_along_axis(decoded[:, None, :, :], idx[..., None], axis=2)

    n = nb * nc
    cs = jnp.full((n, _PRE_PAD), -1e9, jnp.float32)
    cs = cs.at[:, :_PRE_NMS].set(vals.reshape(n, _PRE_NMS))
    cb_soa = jnp.zeros((4, n, _PRE_PAD), jnp.float32)
    cb_soa = cb_soa.at[:, :, :_PRE_NMS].set(
        cb.reshape(n, _PRE_NMS, 4).transpose(2, 0, 1))

    outs, outb = _run_nms(cs, cb_soa)

    out_s = outs[:, :_TOP_K].reshape(nb, nc, _TOP_K, 1)
    out_b = outb[:, :, :_TOP_K].transpose(1, 2, 0).reshape(nb, nc, _TOP_K, 4)
    return jnp.concatenate([out_s, out_b], axis=-1)
